# baseline (speedup 1.0000x reference)
import functools

import jax
import jax.numpy as jnp
from jax import lax
from jax.experimental import pallas as pl
from jax.experimental.pallas import tpu as pltpu
from jax.experimental.pallas import tpu_sc as plsc

N = 10000
K = 16
D = 128
B = 4
OUT = 2

NC, NS, L = 2, 16, 16
NW = NC * NS
ROWW = B * D
HROW = ROWW // 2
NPAD = 10240
CHUNK = 8
NROWS = NPAD * B
CK = CHUNK * K
TOTCH = NPAD // CHUNK
C0 = 24
C1 = 56
C_MAX = max(C0, C1)
assert 16 * (C0 + C1) == TOTCH and C0 % 2 == 0 and C1 % 2 == 0


def _gather_agg_body(x_hbm, adj_hbm, w_hbm, out_hbm, adj_v, rows_a, rows_b,
                     acc_a, acc_b, w_v, sem_a, sem_b, sem_oa, sem_ob):
    cid = lax.axis_index("c")
    sid = lax.axis_index("s")
    nc = jnp.where(cid == 0, C0, C1)
    base_chunk = jnp.where(cid == 0, sid * C0, 16 * C0 + sid * C1)
    wstart = jnp.minimum(base_chunk, TOTCH - C_MAX)
    off = base_chunk - wstart
    pltpu.sync_copy(w_hbm, w_v)
    pltpu.sync_copy(adj_hbm.at[pl.ds(wstart, C_MAX)], adj_v)

    def gather(c, buf, sem):
        return pltpu.async_copy(x_hbm.at[adj_v.at[off + c]], buf, sem)

    def wait_gather(buf, sem):
        pltpu.make_async_copy(x_hbm.at[pl.ds(0, CK)], buf, sem).wait()

    def wait_scatter(acc, sem):
        pltpu.make_async_copy(acc, out_hbm.at[pl.ds(0, CHUNK)], sem).wait()

    def compute(c, buf, acc):
        def vbody(v, carry):
            wv = [w_v[k, pl.ds(v * L, L)] for k in range(K)]
            for j in range(CHUNK):
                r0 = j * K
                for bb in range(B // 2):
                    xi = buf[r0, pl.ds(bb * D + v * L, L)]
                    te = plsc.bitcast(xi << 16, jnp.float32) * wv[0]
                    to = plsc.bitcast(xi & -65536, jnp.float32) * wv[0]
                    for k in range(1, K):
                        xi = buf[r0 + k, pl.ds(bb * D + v * L, L)]
                        te = te + plsc.bitcast(xi << 16, jnp.float32) * wv[k]
                        to = to + plsc.bitcast(xi & -65536, jnp.float32) * wv[k]
                    acc[j, pl.ds(bb * 2 * D + v * L, L)] = te
                    acc[j, pl.ds(bb * 2 * D + D + v * L, L)] = to
            return carry

        lax.fori_loop(0, D // L, vbody, 0)
        return pltpu.async_copy(
            acc, out_hbm.at[pl.ds((base_chunk + c) * CHUNK, CHUNK)],
            sem_oa if acc is acc_a else sem_ob)

    gather(0, rows_a, sem_a)

    def body(t, carry):
        c0 = 2 * t
        c1 = 2 * t + 1
        gather(c1, rows_b, sem_b)
        wait_gather(rows_a, sem_a)

        @pl.when(t > 0)
        def _():
            wait_scatter(acc_a, sem_oa)
        compute(c0, rows_a, acc_a)

        @pl.when(t < nh - 1)
        def _():
            gather(c0 + 2, rows_a, sem_a)
        wait_gather(rows_b, sem_b)

        @pl.when(t > 0)
        def _():
            wait_scatter(acc_b, sem_ob)
        compute(c1, rows_b, acc_b)
        return carry

    nh = nc // 2
    lax.fori_loop(0, nh, body, 0)
    wait_scatter(acc_a, sem_oa)
    wait_scatter(acc_b, sem_ob)


@functools.partial(jax.jit, static_argnames=())
def _gather_agg(xb_rows, adj_w, w_feat):
    mesh = plsc.VectorSubcoreMesh(core_axis_name="c", subcore_axis_name="s")
    return pl.kernel(
        _gather_agg_body,
        out_type=jax.ShapeDtypeStruct((NPAD, ROWW), jnp.float32),
        mesh=mesh,
        compiler_params=pltpu.CompilerParams(needs_layout_passes=False),
        scratch_types=[
            pltpu.VMEM((C_MAX, CK), jnp.int32),
            pltpu.VMEM((CK, HROW), jnp.int32),
            pltpu.VMEM((CK, HROW), jnp.int32),
            pltpu.VMEM((CHUNK, ROWW), jnp.float32),
            pltpu.VMEM((CHUNK, ROWW), jnp.float32),
            pltpu.VMEM((K, D), jnp.float32),
            pltpu.SemaphoreType.DMA,
            pltpu.SemaphoreType.DMA,
            pltpu.SemaphoreType.DMA,
            pltpu.SemaphoreType.DMA,
        ],
    )(xb_rows, adj_w, w_feat)


def _pack_pairs(x):
    u = lax.bitcast_convert_type(x, jnp.uint32)
    r2 = u.shape[0] // 2
    u = u.reshape(r2, 2, u.shape[1])
    one = jnp.uint32(1)
    half = jnp.uint32(0x7FFF)

    def rne(t):
        return (t + half + ((t >> 16) & one)) >> 16

    packed = (rne(u[:, 1, :]) << 16) | rne(u[:, 0, :])
    return lax.bitcast_convert_type(packed, jnp.int32)


def _init_body(lat_ref, win_ref, bin_ref, pos_ref, out_ref, outb_ref):
    x0 = jnp.dot(lat_ref[...], win_ref[...],
                 preferred_element_type=jnp.float32) + bin_ref[...]
    x = pos_ref[...][:, None, :] + x0[None, :, :]
    out_ref[...] = x
    tn = x.shape[0]
    outb_ref[...] = _pack_pairs(x.reshape(tn * B, D)).reshape(tn, B // 2, D)


def _init_x(latent, W_in, b_in, pos_pad):
    tn = 1024
    return pl.pallas_call(
        _init_body,
        grid=(NPAD // tn,),
        in_specs=[
            pl.BlockSpec((B, W_in.shape[0]), lambda i: (0, 0)),
            pl.BlockSpec((W_in.shape[0], D), lambda i: (0, 0)),
            pl.BlockSpec((1, D), lambda i: (0, 0)),
            pl.BlockSpec((tn, D), lambda i: (i, 0)),
        ],
        out_specs=[
            pl.BlockSpec((tn, B, D), lambda i: (i, 0, 0)),
            pl.BlockSpec((tn, B // 2, D), lambda i: (i, 0, 0)),
        ],
        out_shape=[
            jax.ShapeDtypeStruct((NPAD, B, D), jnp.float32),
            jax.ShapeDtypeStruct((NPAD, B // 2, D), jnp.int32),
        ],
    )(latent, W_in, b_in.reshape(1, D), pos_pad)


def _ln(x, g, b):
    m = jnp.mean(x, axis=-1, keepdims=True)
    v = jnp.mean((x - m) ** 2, axis=-1, keepdims=True)
    return (x - m) * lax.rsqrt(v + 1e-5) * g + b


def _block_body(x_ref, agg_ref, g_ref, b_ref, w1_ref, b1_ref, w2_ref, b2_ref,
                out_ref, outb_ref):
    h = _ln(agg_ref[...], g_ref[...], b_ref[...])
    u = jax.nn.gelu(jnp.dot(h, w1_ref[...], preferred_element_type=jnp.float32)
                    + b1_ref[...])
    y = jnp.dot(u, w2_ref[...], preferred_element_type=jnp.float32) + b2_ref[...]
    x = x_ref[...] + y
    out_ref[...] = x
    outb_ref[...] = _pack_pairs(x)


def _mlp_block(x2d, agg2d, g, b, W1, b1, W2, b2):
    r = 2048
    h4 = 4 * D
    return pl.pallas_call(
        _block_body,
        grid=(NROWS // r,),
        in_specs=[
            pl.BlockSpec((r, D), lambda i: (i, 0)),
            pl.BlockSpec((r, D), lambda i: (i, 0)),
            pl.BlockSpec((1, D), lambda i: (0, 0)),
            pl.BlockSpec((1, D), lambda i: (0, 0)),
            pl.BlockSpec((D, h4), lambda i: (0, 0)),
            pl.BlockSpec((1, h4), lambda i: (0, 0)),
            pl.BlockSpec((h4, D), lambda i: (0, 0)),
            pl.BlockSpec((1, D), lambda i: (0, 0)),
        ],
        out_specs=[
            pl.BlockSpec((r, D), lambda i: (i, 0)),
            pl.BlockSpec((r // 2, D), lambda i: (i, 0)),
        ],
        out_shape=[
            jax.ShapeDtypeStruct((NROWS, D), jnp.float32),
            jax.ShapeDtypeStruct((NROWS // 2, D), jnp.int32),
        ],
    )(x2d, agg2d, g.reshape(1, D), b.reshape(1, D), W1, b1.reshape(1, h4),
      W2, b2.reshape(1, D))


def _block_head_body(x_ref, agg_ref, g_ref, b_ref, w1_ref, b1_ref, w2_ref,
                     b2_ref, hg_ref, hb_ref, wh_ref, bh_ref, out_ref):
    h = _ln(agg_ref[...], g_ref[...], b_ref[...])
    u = jax.nn.gelu(jnp.dot(h, w1_ref[...], preferred_element_type=jnp.float32)
                    + b1_ref[...])
    y = jnp.dot(u, w2_ref[...], preferred_element_type=jnp.float32) + b2_ref[...]
    x = x_ref[...] + y
    h2 = _ln(x, hg_ref[...], hb_ref[...])
    out_ref[...] = (jnp.dot(h2, wh_ref[...], preferred_element_type=jnp.float32)
                    + bh_ref[...])


def _mlp_block_head(x2d, agg2d, g, b, W1, b1, W2, b2, hg, hb, W_head, b_head):
    r = 2048
    h4 = 4 * D
    return pl.pallas_call(
        _block_head_body,
        grid=(NROWS // r,),
        in_specs=[
            pl.BlockSpec((r, D), lambda i: (i, 0)),
            pl.BlockSpec((r, D), lambda i: (i, 0)),
            pl.BlockSpec((1, D), lambda i: (0, 0)),
            pl.BlockSpec((1, D), lambda i: (0, 0)),
            pl.BlockSpec((D, h4), lambda i: (0, 0)),
            pl.BlockSpec((1, h4), lambda i: (0, 0)),
            pl.BlockSpec((h4, D), lambda i: (0, 0)),
            pl.BlockSpec((1, D), lambda i: (0, 0)),
            pl.BlockSpec((1, D), lambda i: (0, 0)),
            pl.BlockSpec((1, D), lambda i: (0, 0)),
            pl.BlockSpec((D, OUT), lambda i: (0, 0)),
            pl.BlockSpec((1, OUT), lambda i: (0, 0)),
        ],
        out_specs=pl.BlockSpec((r, OUT), lambda i: (i, 0)),
        out_shape=jax.ShapeDtypeStruct((NROWS, OUT), jnp.float32),
    )(x2d, agg2d, g.reshape(1, D), b.reshape(1, D), W1, b1.reshape(1, h4),
      W2, b2.reshape(1, D), hg.reshape(1, D), hb.reshape(1, D), W_head,
      b_head.reshape(1, OUT))


def kernel(latent_token, adj, W_in, b_in, pos_embed, w_nb, ln1_g, ln1_b,
           W1, b1, W2, b2, lnh_g, lnh_b, W_head, b_head):
    depth = w_nb.shape[0]
    pos_pad = jnp.zeros((NPAD, D), jnp.float32).at[:N].set(pos_embed[0])
    adj_flat = jnp.zeros((NPAD, K), jnp.int32).at[:N].set(
        adj.astype(jnp.int32)).reshape(TOTCH, CK)
    x, xb = _init_x(latent_token, W_in, b_in, pos_pad)
    x = x.reshape(NPAD, ROWW)
    xb = xb.reshape(NPAD, HROW)
    y = None
    for i in range(depth):
        agg = _gather_agg(xb, adj_flat, w_nb[i].astype(jnp.float32))
        if i < depth - 1:
            x2, xb2 = _mlp_block(x.reshape(NROWS, D), agg.reshape(NROWS, D),
                                 ln1_g[i], ln1_b[i], W1[i], b1[i], W2[i], b2[i])
            x = x2.reshape(NPAD, ROWW)
            xb = xb2.reshape(NPAD, HROW)
        else:
            y = _mlp_block_head(x.reshape(NROWS, D), agg.reshape(NROWS, D),
                                ln1_g[i], ln1_b[i], W1[i], b1[i], W2[i], b2[i],
                                lnh_g, lnh_b, W_head, b_head)
    out = y.reshape(NPAD, B, OUT)[:N]
    return jnp.transpose(out, (1, 2, 0))


# speedup vs baseline: 1.2195x; 1.2195x over previous
import functools

import jax
import jax.numpy as jnp
from jax import lax
from jax.experimental import pallas as pl
from jax.experimental.pallas import tpu as pltpu
from jax.experimental.pallas import tpu_sc as plsc

N = 10000
K = 16
D = 128
B = 4
OUT = 2

NC, NS, L = 2, 16, 16
NW = NC * NS
ROWW = B * D
HROW = ROWW // 2
NPAD = 10240
CHUNK = 8
NROWS = NPAD * B
CK = CHUNK * K
TOTCH = NPAD // CHUNK
C0 = 40
C1 = 40
C_MAX = max(C0, C1)
assert 16 * (C0 + C1) == TOTCH and C0 % 2 == 0 and C1 % 2 == 0


def _gather_agg_body(x_hbm, adj_hbm, w_hbm, out_hbm, adj_v, rows_a, rows_b,
                     acc_a, acc_b, w_v, sem_a, sem_b, sem_oa, sem_ob):
    cid = lax.axis_index("c")
    sid = lax.axis_index("s")
    nc = jnp.where(cid == 0, C0, C1)
    base_chunk = jnp.where(cid == 0, sid * C0, 16 * C0 + sid * C1)
    wstart = jnp.minimum(base_chunk, TOTCH - C_MAX)
    off = base_chunk - wstart
    pltpu.sync_copy(w_hbm, w_v)
    pltpu.sync_copy(adj_hbm.at[pl.ds(wstart, C_MAX)], adj_v)

    def gather(c, buf, sem):
        return pltpu.async_copy(x_hbm.at[adj_v.at[off + c]], buf, sem)

    def wait_gather(buf, sem):
        pltpu.make_async_copy(x_hbm.at[pl.ds(0, CK)], buf, sem).wait()

    def wait_scatter(acc, sem):
        pltpu.make_async_copy(acc, out_hbm.at[pl.ds(0, CHUNK)], sem).wait()

    def compute(c, buf, acc):
        def vbody(v, carry):
            wv = [w_v[k, pl.ds(v * L, L)] for k in range(K)]
            for j in range(CHUNK):
                r0 = j * K
                for bb in range(B // 2):
                    xi = buf[r0, pl.ds(bb * D + v * L, L)]
                    te = plsc.bitcast(xi << 16, jnp.float32) * wv[0]
                    to = plsc.bitcast(xi & -65536, jnp.float32) * wv[0]
                    for k in range(1, K):
                        xi = buf[r0 + k, pl.ds(bb * D + v * L, L)]
                        te = te + plsc.bitcast(xi << 16, jnp.float32) * wv[k]
                        to = to + plsc.bitcast(xi & -65536, jnp.float32) * wv[k]
                    acc[j, pl.ds(bb * 2 * D + v * L, L)] = te
                    acc[j, pl.ds(bb * 2 * D + D + v * L, L)] = to
            return carry

        lax.fori_loop(0, D // L, vbody, 0)
        return pltpu.async_copy(
            acc, out_hbm.at[pl.ds((base_chunk + c) * CHUNK, CHUNK)],
            sem_oa if acc is acc_a else sem_ob)

    gather(0, rows_a, sem_a)

    def body(t, carry):
        c0 = 2 * t
        c1 = 2 * t + 1
        gather(c1, rows_b, sem_b)
        wait_gather(rows_a, sem_a)

        @pl.when(t > 0)
        def _():
            wait_scatter(acc_a, sem_oa)
        compute(c0, rows_a, acc_a)

        @pl.when(t < nh - 1)
        def _():
            gather(c0 + 2, rows_a, sem_a)
        wait_gather(rows_b, sem_b)

        @pl.when(t > 0)
        def _():
            wait_scatter(acc_b, sem_ob)
        compute(c1, rows_b, acc_b)
        return carry

    nh = nc // 2
    lax.fori_loop(0, nh, body, 0)
    wait_scatter(acc_a, sem_oa)
    wait_scatter(acc_b, sem_ob)


def _gather_pos_body(x_hbm, adj_hbm, w_hbm, out_hbm, adj_v, rows_a, rows_b,
                     acc_a, acc_b, w_v, sem_a, sem_b, sem_oa, sem_ob):
    cid = lax.axis_index("c")
    sid = lax.axis_index("s")
    nc = jnp.where(cid == 0, C0, C1)
    base_chunk = jnp.where(cid == 0, sid * C0, 16 * C0 + sid * C1)
    wstart = jnp.minimum(base_chunk, TOTCH - C_MAX)
    off = base_chunk - wstart
    pltpu.sync_copy(w_hbm, w_v)
    pltpu.sync_copy(adj_hbm.at[pl.ds(wstart, C_MAX)], adj_v)

    def gather(c, buf, sem):
        return pltpu.async_copy(x_hbm.at[adj_v.at[off + c]], buf, sem)

    def wait_gather(buf, sem):
        pltpu.make_async_copy(x_hbm.at[pl.ds(0, CK)], buf, sem).wait()

    def wait_scatter(acc, sem):
        pltpu.make_async_copy(acc, out_hbm.at[pl.ds(0, CHUNK)], sem).wait()

    def compute(c, buf, acc):
        def vbody(v, carry):
            sl = pl.ds(v * L, L)
            wv = [w_v[k, sl] for k in range(K)]
            for j in range(CHUNK):
                r0 = j * K
                t = buf[r0, sl] * wv[0]
                for k in range(1, K):
                    t = t + buf[r0 + k, sl] * wv[k]
                acc[j, sl] = t
            return carry

        lax.fori_loop(0, D // L, vbody, 0)
        return pltpu.async_copy(
            acc, out_hbm.at[pl.ds((base_chunk + c) * CHUNK, CHUNK)],
            sem_oa if acc is acc_a else sem_ob)

    gather(0, rows_a, sem_a)
    nh = nc // 2

    def body(t, carry):
        c0 = 2 * t
        c1 = 2 * t + 1
        gather(c1, rows_b, sem_b)
        wait_gather(rows_a, sem_a)

        @pl.when(t > 0)
        def _():
            wait_scatter(acc_a, sem_oa)
        compute(c0, rows_a, acc_a)

        @pl.when(t < nh - 1)
        def _():
            gather(c0 + 2, rows_a, sem_a)
        wait_gather(rows_b, sem_b)

        @pl.when(t > 0)
        def _():
            wait_scatter(acc_b, sem_ob)
        compute(c1, rows_b, acc_b)
        return carry

    lax.fori_loop(0, nh, body, 0)
    wait_scatter(acc_a, sem_oa)
    wait_scatter(acc_b, sem_ob)


@functools.partial(jax.jit, static_argnames=())
def _gather_pos(pos2d, adj_w, w_feat):
    mesh = plsc.VectorSubcoreMesh(core_axis_name="c", subcore_axis_name="s")
    return pl.kernel(
        _gather_pos_body,
        out_type=jax.ShapeDtypeStruct((NPAD, D), jnp.float32),
        mesh=mesh,
        compiler_params=pltpu.CompilerParams(needs_layout_passes=False),
        scratch_types=[
            pltpu.VMEM((C_MAX, CK), jnp.int32),
            pltpu.VMEM((CK, D), jnp.float32),
            pltpu.VMEM((CK, D), jnp.float32),
            pltpu.VMEM((CHUNK, D), jnp.float32),
            pltpu.VMEM((CHUNK, D), jnp.float32),
            pltpu.VMEM((K, D), jnp.float32),
            pltpu.SemaphoreType.DMA,
            pltpu.SemaphoreType.DMA,
            pltpu.SemaphoreType.DMA,
            pltpu.SemaphoreType.DMA,
        ],
    )(pos2d, adj_w, w_feat)


@functools.partial(jax.jit, static_argnames=())
def _gather_agg(xb_rows, adj_w, w_feat):
    mesh = plsc.VectorSubcoreMesh(core_axis_name="c", subcore_axis_name="s")
    return pl.kernel(
        _gather_agg_body,
        out_type=jax.ShapeDtypeStruct((NPAD, ROWW), jnp.float32),
        mesh=mesh,
        compiler_params=pltpu.CompilerParams(needs_layout_passes=False),
        scratch_types=[
            pltpu.VMEM((C_MAX, CK), jnp.int32),
            pltpu.VMEM((CK, HROW), jnp.int32),
            pltpu.VMEM((CK, HROW), jnp.int32),
            pltpu.VMEM((CHUNK, ROWW), jnp.float32),
            pltpu.VMEM((CHUNK, ROWW), jnp.float32),
            pltpu.VMEM((K, D), jnp.float32),
            pltpu.SemaphoreType.DMA,
            pltpu.SemaphoreType.DMA,
            pltpu.SemaphoreType.DMA,
            pltpu.SemaphoreType.DMA,
        ],
    )(xb_rows, adj_w, w_feat)


def _pack_pairs(x):
    u = lax.bitcast_convert_type(x, jnp.uint32)
    r2 = u.shape[0] // 2
    u = u.reshape(r2, 2, u.shape[1])
    one = jnp.uint32(1)
    half = jnp.uint32(0x7FFF)

    def rne(t):
        return (t + half + ((t >> 16) & one)) >> 16

    packed = (rne(u[:, 1, :]) << 16) | rne(u[:, 0, :])
    return lax.bitcast_convert_type(packed, jnp.int32)


def _ln(x, g, b):
    m = jnp.mean(x, axis=-1, keepdims=True)
    v = jnp.mean((x - m) ** 2, axis=-1, keepdims=True)
    return (x - m) * lax.rsqrt(v + 1e-5) * g + b


def _block0_body(pos_ref, lat_ref, win_ref, bin_ref, w0_ref, agg0_ref, g_ref,
                 b_ref, w1_ref, b1_ref, w2_ref, b2_ref, out_ref, outb_ref):
    x0 = jnp.dot(lat_ref[...], win_ref[...],
                 preferred_element_type=jnp.float32) + bin_ref[...]
    sw = jnp.sum(w0_ref[...], axis=0, keepdims=True)
    tn = agg0_ref.shape[0]
    a = agg0_ref[...][:, None, :] + (x0 * sw)[None, :, :]
    h = _ln(a.reshape(tn * B, D), g_ref[...], b_ref[...])
    u = jax.nn.gelu(jnp.dot(h, w1_ref[...], preferred_element_type=jnp.float32)
                    + b1_ref[...])
    y = jnp.dot(u, w2_ref[...], preferred_element_type=jnp.float32) + b2_ref[...]
    xres = pos_ref[...][:, None, :] + x0[None, :, :]
    x = xres.reshape(tn * B, D) + y
    out_ref[...] = x
    outb_ref[...] = _pack_pairs(x)


def _tc_block0(pos2d, latent, W_in, b_in, w0, agg0, g, b, W1, b1, W2, b2):
    r = 2048
    tn = r // B
    h4 = 4 * D
    e = W_in.shape[0]
    return pl.pallas_call(
        _block0_body,
        grid=(NROWS // r,),
        in_specs=[
            pl.BlockSpec((tn, D), lambda i: (i, 0)),
            pl.BlockSpec((B, e), lambda i: (0, 0)),
            pl.BlockSpec((e, D), lambda i: (0, 0)),
            pl.BlockSpec((1, D), lambda i: (0, 0)),
            pl.BlockSpec((K, D), lambda i: (0, 0)),
            pl.BlockSpec((tn, D), lambda i: (i, 0)),
            pl.BlockSpec((1, D), lambda i: (0, 0)),
            pl.BlockSpec((1, D), lambda i: (0, 0)),
            pl.BlockSpec((D, h4), lambda i: (0, 0)),
            pl.BlockSpec((1, h4), lambda i: (0, 0)),
            pl.BlockSpec((h4, D), lambda i: (0, 0)),
            pl.BlockSpec((1, D), lambda i: (0, 0)),
        ],
        out_specs=[
            pl.BlockSpec((r, D), lambda i: (i, 0)),
            pl.BlockSpec((r // 2, D), lambda i: (i, 0)),
        ],
        out_shape=[
            jax.ShapeDtypeStruct((NROWS, D), jnp.float32),
            jax.ShapeDtypeStruct((NROWS // 2, D), jnp.int32),
        ],
    )(pos2d, latent, W_in, b_in.reshape(1, D), w0, agg0, g.reshape(1, D),
      b.reshape(1, D), W1, b1.reshape(1, h4), W2, b2.reshape(1, D))


def _block_body(x_ref, agg_ref, g_ref, b_ref, w1_ref, b1_ref, w2_ref, b2_ref,
                out_ref, outb_ref):
    h = _ln(agg_ref[...], g_ref[...], b_ref[...])
    u = jax.nn.gelu(jnp.dot(h, w1_ref[...], preferred_element_type=jnp.float32)
                    + b1_ref[...])
    y = jnp.dot(u, w2_ref[...], preferred_element_type=jnp.float32) + b2_ref[...]
    x = x_ref[...] + y
    out_ref[...] = x
    outb_ref[...] = _pack_pairs(x)


def _mlp_block(x2d, agg2d, g, b, W1, b1, W2, b2):
    r = 2048
    h4 = 4 * D
    return pl.pallas_call(
        _block_body,
        grid=(NROWS // r,),
        in_specs=[
            pl.BlockSpec((r, D), lambda i: (i, 0)),
            pl.BlockSpec((r, D), lambda i: (i, 0)),
            pl.BlockSpec((1, D), lambda i: (0, 0)),
            pl.BlockSpec((1, D), lambda i: (0, 0)),
            pl.BlockSpec((D, h4), lambda i: (0, 0)),
            pl.BlockSpec((1, h4), lambda i: (0, 0)),
            pl.BlockSpec((h4, D), lambda i: (0, 0)),
            pl.BlockSpec((1, D), lambda i: (0, 0)),
        ],
        out_specs=[
            pl.BlockSpec((r, D), lambda i: (i, 0)),
            pl.BlockSpec((r // 2, D), lambda i: (i, 0)),
        ],
        out_shape=[
            jax.ShapeDtypeStruct((NROWS, D), jnp.float32),
            jax.ShapeDtypeStruct((NROWS // 2, D), jnp.int32),
        ],
    )(x2d, agg2d, g.reshape(1, D), b.reshape(1, D), W1, b1.reshape(1, h4),
      W2, b2.reshape(1, D))


def _block_head_body(x_ref, agg_ref, g_ref, b_ref, w1_ref, b1_ref, w2_ref,
                     b2_ref, hg_ref, hb_ref, wh_ref, bh_ref, out_ref):
    h = _ln(agg_ref[...], g_ref[...], b_ref[...])
    u = jax.nn.gelu(jnp.dot(h, w1_ref[...], preferred_element_type=jnp.float32)
                    + b1_ref[...])
    y = jnp.dot(u, w2_ref[...], preferred_element_type=jnp.float32) + b2_ref[...]
    x = x_ref[...] + y
    h2 = _ln(x, hg_ref[...], hb_ref[...])
    out_ref[...] = (jnp.dot(h2, wh_ref[...], preferred_element_type=jnp.float32)
                    + bh_ref[...])


def _mlp_block_head(x2d, agg2d, g, b, W1, b1, W2, b2, hg, hb, W_head, b_head):
    r = 2048
    h4 = 4 * D
    return pl.pallas_call(
        _block_head_body,
        grid=(NROWS // r,),
        in_specs=[
            pl.BlockSpec((r, D), lambda i: (i, 0)),
            pl.BlockSpec((r, D), lambda i: (i, 0)),
            pl.BlockSpec((1, D), lambda i: (0, 0)),
            pl.BlockSpec((1, D), lambda i: (0, 0)),
            pl.BlockSpec((D, h4), lambda i: (0, 0)),
            pl.BlockSpec((1, h4), lambda i: (0, 0)),
            pl.BlockSpec((h4, D), lambda i: (0, 0)),
            pl.BlockSpec((1, D), lambda i: (0, 0)),
            pl.BlockSpec((1, D), lambda i: (0, 0)),
            pl.BlockSpec((1, D), lambda i: (0, 0)),
            pl.BlockSpec((D, OUT), lambda i: (0, 0)),
            pl.BlockSpec((1, OUT), lambda i: (0, 0)),
        ],
        out_specs=pl.BlockSpec((r, OUT), lambda i: (i, 0)),
        out_shape=jax.ShapeDtypeStruct((NROWS, OUT), jnp.float32),
    )(x2d, agg2d, g.reshape(1, D), b.reshape(1, D), W1, b1.reshape(1, h4),
      W2, b2.reshape(1, D), hg.reshape(1, D), hb.reshape(1, D), W_head,
      b_head.reshape(1, OUT))


def kernel(latent_token, adj, W_in, b_in, pos_embed, w_nb, ln1_g, ln1_b,
           W1, b1, W2, b2, lnh_g, lnh_b, W_head, b_head):
    depth = w_nb.shape[0]
    adj_flat = jnp.zeros((NPAD, K), jnp.int32).at[:N].set(
        adj.astype(jnp.int32)).reshape(TOTCH, CK)
    pos2d = pos_embed[0]
    agg0 = _gather_pos(pos2d, adj_flat, w_nb[0].astype(jnp.float32))
    x, xb = _tc_block0(pos2d, latent_token, W_in, b_in,
                       w_nb[0].astype(jnp.float32), agg0,
                       ln1_g[0], ln1_b[0], W1[0], b1[0], W2[0], b2[0])
    y = None
    for i in range(1, depth):
        agg = _gather_agg(xb.reshape(NPAD, HROW), adj_flat,
                          w_nb[i].astype(jnp.float32))
        if i < depth - 1:
            x, xb = _mlp_block(x, agg.reshape(NROWS, D),
                               ln1_g[i], ln1_b[i], W1[i], b1[i], W2[i], b2[i])
        else:
            y = _mlp_block_head(x, agg.reshape(NROWS, D),
                                ln1_g[i], ln1_b[i], W1[i], b1[i], W2[i], b2[i],
                                lnh_g, lnh_b, W_head, b_head)
    out = y.reshape(NPAD, B, OUT)[:N]
    return jnp.transpose(out, (1, 2, 0))


# speedup vs baseline: 1.2667x; 1.0387x over previous
import functools

import jax
import jax.numpy as jnp
from jax import lax
from jax.experimental import pallas as pl
from jax.experimental.pallas import tpu as pltpu
from jax.experimental.pallas import tpu_sc as plsc

N = 10000
K = 16
D = 128
B = 4
OUT = 2

NC, NS, L = 2, 16, 16
NW = NC * NS
ROWW = B * D
HROW = ROWW // 2
NPAD = 10240
CHUNK = 8
NROWS = NPAD * B
CK = CHUNK * K
TOTCH = NPAD // CHUNK
NHALF = NPAD // 2
HALFCH = TOTCH // 2
CSUB = HALFCH // NW
NH = CSUB // 2
WIN = CSUB + 4


def _gather_agg_body(chunk0, x_hbm, adj_hbm, w_hbm, out_hbm, adj_v, rows_a,
                     rows_b, acc_a, acc_b, w_v, sem_a, sem_b, sem_oa, sem_ob):
    cid = lax.axis_index("c")
    sid = lax.axis_index("s")
    base_local = (cid * NS + sid) * CSUB
    wstart = ((chunk0 + base_local) // 8) * 8
    off = chunk0 + base_local - wstart
    pltpu.sync_copy(w_hbm, w_v)
    pltpu.sync_copy(adj_hbm.at[pl.ds(wstart, WIN)], adj_v)

    def gather(c, buf, sem):
        return pltpu.async_copy(x_hbm.at[adj_v.at[off + c]], buf, sem)

    def wait_gather(buf, sem):
        pltpu.make_async_copy(x_hbm.at[pl.ds(0, CK)], buf, sem).wait()

    def wait_scatter(acc, sem):
        pltpu.make_async_copy(acc, out_hbm.at[pl.ds(0, CHUNK)], sem).wait()

    def compute(c, buf, acc):
        def vbody(v, carry):
            wv = [w_v[k, pl.ds(v * L, L)] for k in range(K)]
            for j in range(CHUNK):
                r0 = j * K
                for bb in range(B // 2):
                    xi = buf[r0, pl.ds(bb * D + v * L, L)]
                    te = plsc.bitcast(xi << 16, jnp.float32) * wv[0]
                    to = plsc.bitcast(xi & -65536, jnp.float32) * wv[0]
                    for k in range(1, K):
                        xi = buf[r0 + k, pl.ds(bb * D + v * L, L)]
                        te = te + plsc.bitcast(xi << 16, jnp.float32) * wv[k]
                        to = to + plsc.bitcast(xi & -65536, jnp.float32) * wv[k]
                    acc[j, pl.ds(bb * 2 * D + v * L, L)] = te
                    acc[j, pl.ds(bb * 2 * D + D + v * L, L)] = to
            return carry

        lax.fori_loop(0, D // L, vbody, 0)
        return pltpu.async_copy(
            acc, out_hbm.at[pl.ds((base_local + c) * CHUNK, CHUNK)],
            sem_oa if acc is acc_a else sem_ob)

    gather(0, rows_a, sem_a)

    def body(t, carry):
        c0 = 2 * t
        c1 = 2 * t + 1
        gather(c1, rows_b, sem_b)
        wait_gather(rows_a, sem_a)

        @pl.when(t > 0)
        def _():
            wait_scatter(acc_a, sem_oa)
        compute(c0, rows_a, acc_a)

        @pl.when(t < NH - 1)
        def _():
            gather(c0 + 2, rows_a, sem_a)
        wait_gather(rows_b, sem_b)

        @pl.when(t > 0)
        def _():
            wait_scatter(acc_b, sem_ob)
        compute(c1, rows_b, acc_b)
        return carry

    lax.fori_loop(0, NH, body, 0)
    wait_scatter(acc_a, sem_oa)
    wait_scatter(acc_b, sem_ob)


def _gather_pos_body(chunk0, x_hbm, adj_hbm, w_hbm, out_hbm, adj_v, rows_a,
                     rows_b, acc_a, acc_b, w_v, sem_a, sem_b, sem_oa, sem_ob):
    cid = lax.axis_index("c")
    sid = lax.axis_index("s")
    base_local = (cid * NS + sid) * CSUB
    wstart = ((chunk0 + base_local) // 8) * 8
    off = chunk0 + base_local - wstart
    pltpu.sync_copy(w_hbm, w_v)
    pltpu.sync_copy(adj_hbm.at[pl.ds(wstart, WIN)], adj_v)

    def gather(c, buf, sem):
        return pltpu.async_copy(x_hbm.at[adj_v.at[off + c]], buf, sem)

    def wait_gather(buf, sem):
        pltpu.make_async_copy(x_hbm.at[pl.ds(0, CK)], buf, sem).wait()

    def wait_scatter(acc, sem):
        pltpu.make_async_copy(acc, out_hbm.at[pl.ds(0, CHUNK)], sem).wait()

    def compute(c, buf, acc):
        def vbody(v, carry):
            sl = pl.ds(v * L, L)
            wv = [w_v[k, sl] for k in range(K)]
            for j in range(CHUNK):
                r0 = j * K
                t = buf[r0, sl] * wv[0]
                for k in range(1, K):
                    t = t + buf[r0 + k, sl] * wv[k]
                acc[j, sl] = t
            return carry

        lax.fori_loop(0, D // L, vbody, 0)
        return pltpu.async_copy(
            acc, out_hbm.at[pl.ds((base_local + c) * CHUNK, CHUNK)],
            sem_oa if acc is acc_a else sem_ob)

    gather(0, rows_a, sem_a)

    def body(t, carry):
        c0 = 2 * t
        c1 = 2 * t + 1
        gather(c1, rows_b, sem_b)
        wait_gather(rows_a, sem_a)

        @pl.when(t > 0)
        def _():
            wait_scatter(acc_a, sem_oa)
        compute(c0, rows_a, acc_a)

        @pl.when(t < NH - 1)
        def _():
            gather(c0 + 2, rows_a, sem_a)
        wait_gather(rows_b, sem_b)

        @pl.when(t > 0)
        def _():
            wait_scatter(acc_b, sem_ob)
        compute(c1, rows_b, acc_b)
        return carry

    lax.fori_loop(0, NH, body, 0)
    wait_scatter(acc_a, sem_oa)
    wait_scatter(acc_b, sem_ob)


def _gather_pos(pos2d, adj_w, w_feat, half):
    mesh = plsc.VectorSubcoreMesh(core_axis_name="c", subcore_axis_name="s")
    return pl.kernel(
        functools.partial(_gather_pos_body, half * HALFCH),
        out_type=jax.ShapeDtypeStruct((NHALF, D), jnp.float32),
        mesh=mesh,
        compiler_params=pltpu.CompilerParams(needs_layout_passes=False),
        scratch_types=[
            pltpu.VMEM((WIN, CK), jnp.int32),
            pltpu.VMEM((CK, D), jnp.float32),
            pltpu.VMEM((CK, D), jnp.float32),
            pltpu.VMEM((CHUNK, D), jnp.float32),
            pltpu.VMEM((CHUNK, D), jnp.float32),
            pltpu.VMEM((K, D), jnp.float32),
            pltpu.SemaphoreType.DMA,
            pltpu.SemaphoreType.DMA,
            pltpu.SemaphoreType.DMA,
            pltpu.SemaphoreType.DMA,
        ],
    )(pos2d, adj_w, w_feat)


def _gather_agg(xb_rows, adj_w, w_feat, half):
    mesh = plsc.VectorSubcoreMesh(core_axis_name="c", subcore_axis_name="s")
    return pl.kernel(
        functools.partial(_gather_agg_body, half * HALFCH),
        out_type=jax.ShapeDtypeStruct((NHALF, ROWW), jnp.float32),
        mesh=mesh,
        compiler_params=pltpu.CompilerParams(needs_layout_passes=False),
        scratch_types=[
            pltpu.VMEM((WIN, CK), jnp.int32),
            pltpu.VMEM((CK, HROW), jnp.int32),
            pltpu.VMEM((CK, HROW), jnp.int32),
            pltpu.VMEM((CHUNK, ROWW), jnp.float32),
            pltpu.VMEM((CHUNK, ROWW), jnp.float32),
            pltpu.VMEM((K, D), jnp.float32),
            pltpu.SemaphoreType.DMA,
            pltpu.SemaphoreType.DMA,
            pltpu.SemaphoreType.DMA,
            pltpu.SemaphoreType.DMA,
        ],
    )(xb_rows, adj_w, w_feat)


def _pack_pairs(x):
    u = lax.bitcast_convert_type(x, jnp.uint32)
    r2 = u.shape[0] // 2
    u = u.reshape(r2, 2, u.shape[1])
    one = jnp.uint32(1)
    half = jnp.uint32(0x7FFF)

    def rne(t):
        return (t + half + ((t >> 16) & one)) >> 16

    packed = (rne(u[:, 1, :]) << 16) | rne(u[:, 0, :])
    return lax.bitcast_convert_type(packed, jnp.int32)


def _ln(x, g, b):
    m = jnp.mean(x, axis=-1, keepdims=True)
    v = jnp.mean((x - m) ** 2, axis=-1, keepdims=True)
    return (x - m) * lax.rsqrt(v + 1e-5) * g + b


def _block0_body(pos_ref, lat_ref, win_ref, bin_ref, w0_ref, agg0_ref, g_ref,
                 b_ref, w1_ref, b1_ref, w2_ref, b2_ref, out_ref, outb_ref):
    x0 = jnp.dot(lat_ref[...], win_ref[...],
                 preferred_element_type=jnp.float32) + bin_ref[...]
    sw = jnp.sum(w0_ref[...], axis=0, keepdims=True)
    tn = agg0_ref.shape[0]
    a = agg0_ref[...][:, None, :] + (x0 * sw)[None, :, :]
    h = _ln(a.reshape(tn * B, D), g_ref[...], b_ref[...])
    u = jax.nn.gelu(jnp.dot(h, w1_ref[...], preferred_element_type=jnp.float32)
                    + b1_ref[...])
    y = jnp.dot(u, w2_ref[...], preferred_element_type=jnp.float32) + b2_ref[...]
    xres = pos_ref[...][:, None, :] + x0[None, :, :]
    x = xres.reshape(tn * B, D) + y
    out_ref[...] = x
    outb_ref[...] = _pack_pairs(x)


def _tc_block0(pos2d, latent, W_in, b_in, w0, agg0, g, b, W1, b1, W2, b2,
               half):
    r = 2048
    tn = r // B
    h4 = 4 * D
    e = W_in.shape[0]
    rows = agg0.shape[0] * B
    blk0 = half * (NHALF // tn)
    return pl.pallas_call(
        _block0_body,
        grid=(rows // r,),
        in_specs=[
            pl.BlockSpec((tn, D), lambda i: (i + blk0, 0)),
            pl.BlockSpec((B, e), lambda i: (0, 0)),
            pl.BlockSpec((e, D), lambda i: (0, 0)),
            pl.BlockSpec((1, D), lambda i: (0, 0)),
            pl.BlockSpec((K, D), lambda i: (0, 0)),
            pl.BlockSpec((tn, D), lambda i: (i, 0)),
            pl.BlockSpec((1, D), lambda i: (0, 0)),
            pl.BlockSpec((1, D), lambda i: (0, 0)),
            pl.BlockSpec((D, h4), lambda i: (0, 0)),
            pl.BlockSpec((1, h4), lambda i: (0, 0)),
            pl.BlockSpec((h4, D), lambda i: (0, 0)),
            pl.BlockSpec((1, D), lambda i: (0, 0)),
        ],
        out_specs=[
            pl.BlockSpec((r, D), lambda i: (i, 0)),
            pl.BlockSpec((r // 2, D), lambda i: (i, 0)),
        ],
        out_shape=[
            jax.ShapeDtypeStruct((rows, D), jnp.float32),
            jax.ShapeDtypeStruct((rows // 2, D), jnp.int32),
        ],
    )(pos2d, latent, W_in, b_in.reshape(1, D), w0, agg0, g.reshape(1, D),
      b.reshape(1, D), W1, b1.reshape(1, h4), W2, b2.reshape(1, D))


def _block_body(x_ref, agg_ref, g_ref, b_ref, w1_ref, b1_ref, w2_ref, b2_ref,
                out_ref, outb_ref):
    h = _ln(agg_ref[...], g_ref[...], b_ref[...])
    u = jax.nn.gelu(jnp.dot(h, w1_ref[...], preferred_element_type=jnp.float32)
                    + b1_ref[...])
    y = jnp.dot(u, w2_ref[...], preferred_element_type=jnp.float32) + b2_ref[...]
    x = x_ref[...] + y
    out_ref[...] = x
    outb_ref[...] = _pack_pairs(x)


def _mlp_block(x2d, agg2d, g, b, W1, b1, W2, b2):
    r = 2048
    h4 = 4 * D
    rows = x2d.shape[0]
    return pl.pallas_call(
        _block_body,
        grid=(rows // r,),
        in_specs=[
            pl.BlockSpec((r, D), lambda i: (i, 0)),
            pl.BlockSpec((r, D), lambda i: (i, 0)),
            pl.BlockSpec((1, D), lambda i: (0, 0)),
            pl.BlockSpec((1, D), lambda i: (0, 0)),
            pl.BlockSpec((D, h4), lambda i: (0, 0)),
            pl.BlockSpec((1, h4), lambda i: (0, 0)),
            pl.BlockSpec((h4, D), lambda i: (0, 0)),
            pl.BlockSpec((1, D), lambda i: (0, 0)),
        ],
        out_specs=[
            pl.BlockSpec((r, D), lambda i: (i, 0)),
            pl.BlockSpec((r // 2, D), lambda i: (i, 0)),
        ],
        out_shape=[
            jax.ShapeDtypeStruct((rows, D), jnp.float32),
            jax.ShapeDtypeStruct((rows // 2, D), jnp.int32),
        ],
    )(x2d, agg2d, g.reshape(1, D), b.reshape(1, D), W1, b1.reshape(1, h4),
      W2, b2.reshape(1, D))


def _block_head_body(x_ref, agg_ref, g_ref, b_ref, w1_ref, b1_ref, w2_ref,
                     b2_ref, hg_ref, hb_ref, wh_ref, bh_ref, out_ref):
    h = _ln(agg_ref[...], g_ref[...], b_ref[...])
    u = jax.nn.gelu(jnp.dot(h, w1_ref[...], preferred_element_type=jnp.float32)
                    + b1_ref[...])
    y = jnp.dot(u, w2_ref[...], preferred_element_type=jnp.float32) + b2_ref[...]
    x = x_ref[...] + y
    h2 = _ln(x, hg_ref[...], hb_ref[...])
    out_ref[...] = (jnp.dot(h2, wh_ref[...], preferred_element_type=jnp.float32)
                    + bh_ref[...])


def _mlp_block_head(x2d, agg2d, g, b, W1, b1, W2, b2, hg, hb, W_head, b_head):
    r = 2048
    h4 = 4 * D
    rows = x2d.shape[0]
    return pl.pallas_call(
        _block_head_body,
        grid=(rows // r,),
        in_specs=[
            pl.BlockSpec((r, D), lambda i: (i, 0)),
            pl.BlockSpec((r, D), lambda i: (i, 0)),
            pl.BlockSpec((1, D), lambda i: (0, 0)),
            pl.BlockSpec((1, D), lambda i: (0, 0)),
            pl.BlockSpec((D, h4), lambda i: (0, 0)),
            pl.BlockSpec((1, h4), lambda i: (0, 0)),
            pl.BlockSpec((h4, D), lambda i: (0, 0)),
            pl.BlockSpec((1, D), lambda i: (0, 0)),
            pl.BlockSpec((1, D), lambda i: (0, 0)),
            pl.BlockSpec((1, D), lambda i: (0, 0)),
            pl.BlockSpec((D, OUT), lambda i: (0, 0)),
            pl.BlockSpec((1, OUT), lambda i: (0, 0)),
        ],
        out_specs=pl.BlockSpec((r, OUT), lambda i: (i, 0)),
        out_shape=jax.ShapeDtypeStruct((rows, OUT), jnp.float32),
    )(x2d, agg2d, g.reshape(1, D), b.reshape(1, D), W1, b1.reshape(1, h4),
      W2, b2.reshape(1, D), hg.reshape(1, D), hb.reshape(1, D), W_head,
      b_head.reshape(1, OUT))


def kernel(latent_token, adj, W_in, b_in, pos_embed, w_nb, ln1_g, ln1_b,
           W1, b1, W2, b2, lnh_g, lnh_b, W_head, b_head):
    depth = w_nb.shape[0]
    adj_flat = jnp.zeros((NPAD, K), jnp.int32).at[:N].set(
        adj.astype(jnp.int32)).reshape(TOTCH, CK)
    pos2d = pos_embed[0]
    w0 = w_nb[0].astype(jnp.float32)
    agg0 = [_gather_pos(pos2d, adj_flat, w0, h) for h in range(2)]
    xh, xbh = [None, None], [None, None]
    for h in range(2):
        xh[h], xbh[h] = _tc_block0(
            pos2d, latent_token, W_in, b_in, w0, agg0[h],
            ln1_g[0], ln1_b[0], W1[0], b1[0], W2[0], b2[0], h)
    y = [None, None]
    for i in range(1, depth):
        xb = jnp.concatenate(
            [b.reshape(NHALF, HROW) for b in xbh], axis=0)
        wi = w_nb[i].astype(jnp.float32)
        agg = [_gather_agg(xb, adj_flat, wi, h) for h in range(2)]
        for h in range(2):
            if i < depth - 1:
                xh[h], xbh[h] = _mlp_block(
                    xh[h], agg[h].reshape(NHALF * B, D),
                    ln1_g[i], ln1_b[i], W1[i], b1[i], W2[i], b2[i])
            else:
                y[h] = _mlp_block_head(
                    xh[h], agg[h].reshape(NHALF * B, D),
                    ln1_g[i], ln1_b[i], W1[i], b1[i], W2[i], b2[i],
                    lnh_g, lnh_b, W_head, b_head)
    out = jnp.concatenate(y, axis=0).reshape(NPAD, B, OUT)[:N]
    return jnp.transpose(out, (1, 2, 0))
